# baseline (speedup 1.0000x reference)
import functools

import jax
import jax.numpy as jnp
from jax import lax
from jax.experimental import pallas as pl
from jax.experimental.pallas import tpu as pltpu
from jax.experimental.pallas import tpu_sc as plsc

_NUM_CORES = 2
_NUM_SUBCORES = 16
_NUM_WORKERS = _NUM_CORES * _NUM_SUBCORES
_NBUF = 4


def _gather_body(n_chunks, chunk, table_hbm, idx_hbm, out_hbm,
                 idx_v, rows_v, gsem, wsem):
    wid = lax.axis_index("s") * _NUM_CORES + lax.axis_index("c")
    base = wid * (n_chunks * chunk)

    def load_idx(j, b):
        pltpu.sync_copy(idx_hbm.at[pl.ds(base + j * chunk, chunk)],
                        idx_v.at[b])

    def fire_gather(b):
        pltpu.async_copy(table_hbm.at[idx_v.at[b]], rows_v.at[b],
                         gsem.at[b])

    def wait_gather(b):
        pltpu.make_async_copy(table_hbm.at[idx_v.at[b]], rows_v.at[b],
                              gsem.at[b]).wait()

    def fire_wb(j, b):
        pltpu.async_copy(rows_v.at[b],
                         out_hbm.at[pl.ds(base + j * chunk, chunk),
                                    pl.ds(0, rows_v.shape[2])], wsem.at[b])

    def wait_wb(j, b):
        pltpu.make_async_copy(rows_v.at[b],
                              out_hbm.at[pl.ds(base + j * chunk, chunk),
                                         pl.ds(0, rows_v.shape[2])],
                              wsem.at[b]).wait()

    load_idx(0, 0)
    fire_gather(0)
    load_idx(1, 1)
    fire_gather(1)

    for j in range(4):
        b, bn = j % _NBUF, (j + 2) % _NBUF
        wait_gather(b)
        fire_wb(j, b)
        if j >= 2:
            wait_wb(j - 2, bn)
        load_idx(j + 2, bn)
        fire_gather(bn)

    def step(jo, carry):
        j0 = jo * _NBUF
        for b in range(_NBUF):
            j = j0 + b
            bn = (b + 2) % _NBUF
            wait_gather(b)
            fire_wb(j, b)
            wait_wb(j - 2, bn)
            load_idx(j + 2, bn)
            fire_gather(bn)
        return carry

    lax.fori_loop(1, n_chunks // _NBUF - 1, step, 0)

    for j in range(n_chunks - 4, n_chunks):
        b = j % _NBUF
        wait_gather(b)
        fire_wb(j, b)
        if j + 2 < n_chunks:
            bn = (b + 2) % _NBUF
            wait_wb(j - 2, bn)
            load_idx(j + 2, bn)
            fire_gather(bn)
    for j in range(n_chunks - 4, n_chunks):
        wait_wb(j, j % _NBUF)


@functools.partial(jax.jit, static_argnames=("n_rows", "chunk"))
def _sc_embedding_lookup(idx_flat, table, *, n_rows, chunk):
    width = table.shape[1]
    per_worker = n_rows // _NUM_WORKERS
    n_chunks = per_worker // chunk
    mesh = plsc.VectorSubcoreMesh(
        core_axis_name="c", subcore_axis_name="s",
        num_cores=_NUM_CORES, num_subcores=_NUM_SUBCORES)
    body = functools.partial(_gather_body, n_chunks, chunk)
    return pl.kernel(
        body,
        out_type=jax.ShapeDtypeStruct((n_rows, 128), jnp.float32),
        mesh=mesh,
        scratch_types=[
            pltpu.VMEM((_NBUF, chunk), jnp.int32),
            pltpu.VMEM((_NBUF, chunk, width), jnp.float32),
            pltpu.SemaphoreType.DMA((_NBUF,)),
            pltpu.SemaphoreType.DMA((_NBUF,)),
        ],
        compiler_params=pltpu.CompilerParams(use_tc_tiling_on_sc=False),
    )(table, idx_flat)


def kernel(inputs, embeddings):
    n_rows = inputs.shape[0] * inputs.shape[1]
    width = embeddings.shape[1]
    idx_flat = jnp.reshape(inputs, (n_rows,)).astype(jnp.int32)
    out = _sc_embedding_lookup(idx_flat, embeddings, n_rows=n_rows,
                               chunk=400)
    return jnp.reshape(out[:, :width], inputs.shape + (width,))
